# baseline (speedup 1.0000x reference)
import functools

import jax
import jax.numpy as jnp
import numpy as np
from jax import lax
from jax.experimental import pallas as pl

_FEAT = 128
_NRBF = 20
_CUT = 5.0
_NNODES = 10000
_BLK = 1280
_WIN = 64
_NPAD = 10240


def _edge_kernel(m_ref, h_ref, v_ref, d_ref, u_ref,
                 w1_ref, b1_ref, w2a_ref, b2a_ref, w2b_ref, b2b_ref,
                 w2c_ref, b2c_ref, wda_ref, wdb_ref, wdc_ref,
                 dh_ref, dv_ref, cnt_ref, *, blk):
    pid = pl.program_id(0)

    @pl.when(pid == 0)
    def _init():
        def zero_chunk(i, carry):
            dh_ref[pl.ds(i * _WIN, _WIN), :] = jnp.zeros((_WIN, _FEAT),
                                                         jnp.float32)
            cnt_ref[pl.ds(i * _WIN, _WIN), :] = jnp.zeros((_WIN, 8),
                                                          jnp.float32)
            for a in range(3):
                dv_ref[a, pl.ds(i * _WIN, _WIN), :] = jnp.zeros(
                    (_WIN, _FEAT), jnp.float32)
            return carry
        lax.fori_loop(0, _NPAD // _WIN, zero_chunk, 0)

    h = h_ref[...].astype(jnp.bfloat16)
    s = jax.nn.silu(jnp.dot(h, w1_ref[...].astype(jnp.bfloat16),
                            preferred_element_type=jnp.float32) + b1_ref[...])
    sb = s.astype(jnp.bfloat16)
    phi1 = jnp.dot(sb, w2a_ref[...].astype(jnp.bfloat16),
                   preferred_element_type=jnp.float32) + b2a_ref[...]
    phi2 = jnp.dot(sb, w2b_ref[...].astype(jnp.bfloat16),
                   preferred_element_type=jnp.float32) + b2b_ref[...]
    phi3 = jnp.dot(sb, w2c_ref[...].astype(jnp.bfloat16),
                   preferred_element_type=jnp.float32) + b2c_ref[...]

    d = d_ref[...]
    k = jnp.float32(np.pi / _CUT)
    theta = k * d
    s1 = jnp.sin(theta)
    c1 = jnp.cos(theta)
    fc = 0.5 * (c1 + 1.0) * (d < _CUT).astype(jnp.float32)
    g = fc / d
    rows = [s1 * g]
    s_prev, s_cur = jnp.zeros_like(s1), s1
    for _ in range(_NRBF - 1):
        s_prev, s_cur = s_cur, 2.0 * c1 * s_cur - s_prev
        rows.append(s_cur * g)
    rows.append(fc)
    rbf_t = jnp.concatenate(rows, axis=0)
    dd = (((0,), (0,)), ((), ()))
    demb2 = lax.dot_general(rbf_t, wdb_ref[...], dd,
                            preferred_element_type=jnp.float32)
    demb3 = lax.dot_general(rbf_t, wdc_ref[...], dd,
                            preferred_element_type=jnp.float32)
    u0 = u_ref[0:1, :]
    u1 = u_ref[1:2, :]
    u2 = u_ref[2:3, :]
    demb1u0 = lax.dot_general(rbf_t * u0, wda_ref[...], dd,
                              preferred_element_type=jnp.float32)
    demb1u1 = lax.dot_general(rbf_t * u1, wda_ref[...], dd,
                              preferred_element_type=jnp.float32)
    demb1u2 = lax.dot_general(rbf_t * u2, wda_ref[...], dd,
                              preferred_element_type=jnp.float32)

    f2 = phi2 * demb2
    dh = phi3 * demb3
    dv0 = phi1 * demb1u0 + f2 * v_ref[0]
    dv1 = phi1 * demb1u1 + f2 * v_ref[1]
    dv2 = phi1 * demb1u2 + f2 * v_ref[2]
    x = jnp.concatenate([dh, dv0, dv1, dv2],
                        axis=1).astype(jnp.bfloat16)

    m = m_ref[...]
    first = jnp.min(m)
    last = jnp.max(m)
    w0 = (first // _WIN) * _WIN
    npass = (last // _WIN) - (first // _WIN) + 1
    ones_b = jnp.ones((blk, 8), jnp.bfloat16)

    def scatter_pass(p, carry):
        base = w0 + p * _WIN
        col = lax.broadcasted_iota(jnp.int32, (blk, _WIN), 1) + base
        oh = (col == m).astype(jnp.bfloat16)
        c = lax.dot_general(oh, x, (((0,), (0,)), ((), ())),
                            preferred_element_type=jnp.float32)
        dh_ref[pl.ds(base, _WIN), :] += c[:, :_FEAT]
        for a in range(3):
            dv_ref[a, pl.ds(base, _WIN), :] += (
                c[:, (a + 1) * _FEAT:(a + 2) * _FEAT])
        cc = lax.dot_general(oh, ones_b, (((0,), (0,)), ((), ())),
                             preferred_element_type=jnp.float32)
        cnt_ref[pl.ds(base, _WIN), :] += cc
        return carry
    lax.fori_loop(0, npass, scatter_pass, 0)


def kernel(h_i, v_i, d_iI, unit_r_iI, mapping, W1, b1, W2, b2, Wd, bd):
    e = h_i.shape[0]
    blk = _BLK if e % _BLK == 0 else e
    nblk = e // blk

    w2a = W2[:, :_FEAT]
    w2b = W2[:, _FEAT:2 * _FEAT]
    w2c = W2[:, 2 * _FEAT:]
    b2a = b2[:_FEAT].reshape(1, -1)
    b2b = b2[_FEAT:2 * _FEAT].reshape(1, -1)
    b2c = b2[2 * _FEAT:].reshape(1, -1)
    wda = jnp.concatenate([Wd[:, :_FEAT], bd[:_FEAT].reshape(1, -1)], axis=0)
    wdb = jnp.concatenate([Wd[:, _FEAT:2 * _FEAT],
                           bd[_FEAT:2 * _FEAT].reshape(1, -1)], axis=0)
    wdc = jnp.concatenate([Wd[:, 2 * _FEAT:],
                           bd[2 * _FEAT:].reshape(1, -1)], axis=0)

    m2 = mapping.astype(jnp.int32).reshape(e, 1)

    def bspec(shape):
        return pl.BlockSpec(shape, lambda i: (i, 0))

    def wspec(shape):
        return pl.BlockSpec(shape, lambda i: (0, 0))

    acc_dh, acc_dv, cnt = pl.pallas_call(
        functools.partial(_edge_kernel, blk=blk),
        grid=(nblk,),
        in_specs=[
            bspec((blk, 1)),
            bspec((blk, _FEAT)),
            pl.BlockSpec((3, blk, _FEAT), lambda i: (0, i, 0)),
            pl.BlockSpec((1, blk), lambda i: (0, i)),
            pl.BlockSpec((3, blk), lambda i: (0, i)),
            wspec((_FEAT, _FEAT)), wspec((1, _FEAT)),
            wspec((_FEAT, _FEAT)), wspec((1, _FEAT)),
            wspec((_FEAT, _FEAT)), wspec((1, _FEAT)),
            wspec((_FEAT, _FEAT)), wspec((1, _FEAT)),
            wspec((_NRBF + 1, _FEAT)),
            wspec((_NRBF + 1, _FEAT)),
            wspec((_NRBF + 1, _FEAT)),
        ],
        out_specs=[
            pl.BlockSpec((_NPAD, _FEAT), lambda i: (0, 0)),
            pl.BlockSpec((3, _NPAD, _FEAT), lambda i: (0, 0, 0)),
            pl.BlockSpec((_NPAD, 8), lambda i: (0, 0)),
        ],
        out_shape=[
            jax.ShapeDtypeStruct((_NPAD, _FEAT), jnp.float32),
            jax.ShapeDtypeStruct((3, _NPAD, _FEAT), jnp.float32),
            jax.ShapeDtypeStruct((_NPAD, 8), jnp.float32),
        ],
    )(m2, h_i, jnp.transpose(v_i, (2, 0, 1)), d_iI.reshape(1, e),
      unit_r_iI.T,
      W1, b1.reshape(1, -1), w2a, b2a, w2b, b2b, w2c, b2c,
      wda, wdb, wdc)

    counts = jnp.maximum(cnt[:_NNODES, :1], 1.0)
    dh_i = acc_dh[:_NNODES, :] / counts
    dv_pl = acc_dv[:, :_NNODES, :] / counts[None, :, :]
    dv_i = jnp.transpose(dv_pl, (1, 2, 0))
    return (dh_i, dv_i)


# speedup vs baseline: 1.3336x; 1.3336x over previous
import functools

import jax
import jax.numpy as jnp
import numpy as np
from jax import lax
from jax.experimental import pallas as pl

_FEAT = 128
_NRBF = 20
_CUT = 5.0
_NNODES = 10000
_BLK = 1280
_WIN = 128
_NPAD = 10240


def _edge_kernel(m_ref, h_ref, v_ref, d_ref, u_ref,
                 w1_ref, b1_ref, w2a_ref, b2a_ref, w2b_ref, b2b_ref,
                 w2c_ref, b2c_ref, wda_ref, wdb_ref, wdc_ref,
                 dh_ref, dv_ref, cnt_ref, *, blk):
    pid = pl.program_id(0)

    @pl.when(pid == 0)
    def _init():
        def zero_chunk(i, carry):
            dh_ref[pl.ds(i * _WIN, _WIN), :] = jnp.zeros((_WIN, _FEAT),
                                                         jnp.float32)
            cnt_ref[pl.ds(i * _WIN, _WIN), :] = jnp.zeros((_WIN, 8),
                                                          jnp.float32)
            for a in range(3):
                dv_ref[a, pl.ds(i * _WIN, _WIN), :] = jnp.zeros(
                    (_WIN, _FEAT), jnp.float32)
            return carry
        lax.fori_loop(0, _NPAD // _WIN, zero_chunk, 0)

    h = h_ref[...].astype(jnp.bfloat16)
    s = jax.nn.silu(jnp.dot(h, w1_ref[...].astype(jnp.bfloat16),
                            preferred_element_type=jnp.float32) + b1_ref[...])
    sb = s.astype(jnp.bfloat16)
    phi1 = jnp.dot(sb, w2a_ref[...].astype(jnp.bfloat16),
                   preferred_element_type=jnp.float32) + b2a_ref[...]
    phi2 = jnp.dot(sb, w2b_ref[...].astype(jnp.bfloat16),
                   preferred_element_type=jnp.float32) + b2b_ref[...]
    phi3 = jnp.dot(sb, w2c_ref[...].astype(jnp.bfloat16),
                   preferred_element_type=jnp.float32) + b2c_ref[...]

    d = d_ref[...]
    k = jnp.float32(np.pi / _CUT)
    theta = k * d
    s1 = jnp.sin(theta)
    c1 = jnp.cos(theta)
    fc = 0.5 * (c1 + 1.0) * (d < _CUT).astype(jnp.float32)
    g = fc / d
    rows = [s1 * g]
    s_prev, s_cur = jnp.zeros_like(s1), s1
    for _ in range(_NRBF - 1):
        s_prev, s_cur = s_cur, 2.0 * c1 * s_cur - s_prev
        rows.append(s_cur * g)
    rows.append(fc)
    rbf_t = jnp.concatenate(rows, axis=0)
    dd = (((0,), (0,)), ((), ()))
    demb2 = lax.dot_general(rbf_t, wdb_ref[...], dd,
                            preferred_element_type=jnp.float32)
    demb3 = lax.dot_general(rbf_t, wdc_ref[...], dd,
                            preferred_element_type=jnp.float32)
    u0 = u_ref[0:1, :]
    u1 = u_ref[1:2, :]
    u2 = u_ref[2:3, :]
    demb1u0 = lax.dot_general(rbf_t * u0, wda_ref[...], dd,
                              preferred_element_type=jnp.float32)
    demb1u1 = lax.dot_general(rbf_t * u1, wda_ref[...], dd,
                              preferred_element_type=jnp.float32)
    demb1u2 = lax.dot_general(rbf_t * u2, wda_ref[...], dd,
                              preferred_element_type=jnp.float32)

    f2 = phi2 * demb2
    dh = phi3 * demb3
    dv0 = phi1 * demb1u0 + f2 * v_ref[0]
    dv1 = phi1 * demb1u1 + f2 * v_ref[1]
    dv2 = phi1 * demb1u2 + f2 * v_ref[2]
    x = jnp.concatenate([dh, dv0, dv1, dv2],
                        axis=1).astype(jnp.bfloat16)

    m = m_ref[...]
    first = jnp.min(m)
    last = jnp.max(m)
    w0 = (first // _WIN) * _WIN
    npass = (last // _WIN) - (first // _WIN) + 1
    ones_b = jnp.ones((blk, 8), jnp.bfloat16)

    def do_pass(base):
        col = lax.broadcasted_iota(jnp.int32, (blk, _WIN), 1) + base
        oh = (col == m).astype(jnp.bfloat16)
        c = lax.dot_general(oh, x, (((0,), (0,)), ((), ())),
                            preferred_element_type=jnp.float32)
        dh_ref[pl.ds(base, _WIN), :] += c[:, :_FEAT]
        for a in range(3):
            dv_ref[a, pl.ds(base, _WIN), :] += (
                c[:, (a + 1) * _FEAT:(a + 2) * _FEAT])
        cc = lax.dot_general(oh, ones_b, (((0,), (0,)), ((), ())),
                             preferred_element_type=jnp.float32)
        cnt_ref[pl.ds(base, _WIN), :] += cc

    do_pass(w0)

    @pl.when(npass > 1)
    def _pass1():
        do_pass(w0 + _WIN)

    def scatter_tail(p, carry):
        do_pass(w0 + p * _WIN)
        return carry
    lax.fori_loop(2, npass, scatter_tail, 0)


def kernel(h_i, v_i, d_iI, unit_r_iI, mapping, W1, b1, W2, b2, Wd, bd):
    e = h_i.shape[0]
    blk = _BLK if e % _BLK == 0 else e
    nblk = e // blk

    w2a = W2[:, :_FEAT]
    w2b = W2[:, _FEAT:2 * _FEAT]
    w2c = W2[:, 2 * _FEAT:]
    b2a = b2[:_FEAT].reshape(1, -1)
    b2b = b2[_FEAT:2 * _FEAT].reshape(1, -1)
    b2c = b2[2 * _FEAT:].reshape(1, -1)
    wda = jnp.concatenate([Wd[:, :_FEAT], bd[:_FEAT].reshape(1, -1)], axis=0)
    wdb = jnp.concatenate([Wd[:, _FEAT:2 * _FEAT],
                           bd[_FEAT:2 * _FEAT].reshape(1, -1)], axis=0)
    wdc = jnp.concatenate([Wd[:, 2 * _FEAT:],
                           bd[2 * _FEAT:].reshape(1, -1)], axis=0)

    m2 = mapping.astype(jnp.int32).reshape(e, 1)

    def bspec(shape):
        return pl.BlockSpec(shape, lambda i: (i, 0))

    def wspec(shape):
        return pl.BlockSpec(shape, lambda i: (0, 0))

    acc_dh, acc_dv, cnt = pl.pallas_call(
        functools.partial(_edge_kernel, blk=blk),
        grid=(nblk,),
        in_specs=[
            bspec((blk, 1)),
            bspec((blk, _FEAT)),
            pl.BlockSpec((3, blk, _FEAT), lambda i: (0, i, 0)),
            pl.BlockSpec((1, blk), lambda i: (0, i)),
            pl.BlockSpec((3, blk), lambda i: (0, i)),
            wspec((_FEAT, _FEAT)), wspec((1, _FEAT)),
            wspec((_FEAT, _FEAT)), wspec((1, _FEAT)),
            wspec((_FEAT, _FEAT)), wspec((1, _FEAT)),
            wspec((_FEAT, _FEAT)), wspec((1, _FEAT)),
            wspec((_NRBF + 1, _FEAT)),
            wspec((_NRBF + 1, _FEAT)),
            wspec((_NRBF + 1, _FEAT)),
        ],
        out_specs=[
            pl.BlockSpec((_NPAD, _FEAT), lambda i: (0, 0)),
            pl.BlockSpec((3, _NPAD, _FEAT), lambda i: (0, 0, 0)),
            pl.BlockSpec((_NPAD, 8), lambda i: (0, 0)),
        ],
        out_shape=[
            jax.ShapeDtypeStruct((_NPAD, _FEAT), jnp.float32),
            jax.ShapeDtypeStruct((3, _NPAD, _FEAT), jnp.float32),
            jax.ShapeDtypeStruct((_NPAD, 8), jnp.float32),
        ],
    )(m2, h_i, jnp.transpose(v_i, (2, 0, 1)), d_iI.reshape(1, e),
      unit_r_iI.T,
      W1, b1.reshape(1, -1), w2a, b2a, w2b, b2b, w2c, b2c,
      wda, wdb, wdc)

    counts = jnp.maximum(cnt[:_NNODES, :1], 1.0)
    dh_i = acc_dh[:_NNODES, :] / counts
    dv_pl = acc_dv[:, :_NNODES, :] / counts[None, :, :]
    dv_i = jnp.transpose(dv_pl, (1, 2, 0))
    return (dh_i, dv_i)
